# baseline (speedup 1.0000x reference)
import jax
import jax.numpy as jnp
from jax import lax
from jax.experimental import pallas as pl
from jax.experimental.pallas import tpu as pltpu
from jax.experimental.pallas import tpu_sc as plsc

B = 16
T = 2048
DIN = 256
HID = 256
K = 16
TT = 2048
NT = T // TT
BIG = 3.0e38
L = 16


def _score_body(feat, para, pW1a, pW1b, pb1, pW2, pb2, fWb, fb,
                fWa, emb, Wc, A, P, sb1, sW2, g, out, pconst, cblk,
                wcat, pcA):
    t = pl.program_id(0)
    b = pl.program_id(1)

    @pl.when((t == 0) & (b == 0))
    def _():
        pv = para[...]
        h1 = (pv[:, 0:1] * pW1a[...][None, :]
              + pv[:, 1:2] * pW1b[...][None, :]
              + pb1[...][None, :])
        pe = (jnp.dot(jnp.maximum(h1, 0.0), pW2[...],
                      preferred_element_type=jnp.float32, precision=jax.lax.Precision.HIGHEST)
              + pb2[...][None, :])
        pc = (jnp.dot(pe, fWb[...], preferred_element_type=jnp.float32, precision=jax.lax.Precision.HIGHEST)
              + fb[...][None, :])
        pconst[...] = pc
        wcat[:, :HID] = fWa[...]
        wcat[:, HID:] = jnp.dot(fWa[...], A[...],
                                preferred_element_type=jnp.float32, precision=jax.lax.Precision.HIGHEST)
        pcA[...] = jnp.dot(pc, A[...], preferred_element_type=jnp.float32, precision=jax.lax.Precision.HIGHEST)

    @pl.when(b == 0)
    def _():
        cblk[...] = (jnp.dot(emb[...], Wc[...],
                             preferred_element_type=jnp.float32, precision=jax.lax.Precision.HIGHEST)
                     + sb1[...][None, :])

    x = feat[0]
    r = jnp.dot(x, wcat[...], preferred_element_type=jnp.float32, precision=jax.lax.Precision.HIGHEST)
    fpm = r[:, :HID] + pconst[pl.ds(b, 1), :]
    prod = fpm * emb[...]
    pre = (jnp.dot(prod, P[...], preferred_element_type=jnp.float32, precision=jax.lax.Precision.HIGHEST)
           + r[:, HID:] + pcA[pl.ds(b, 1), :] + cblk[...])
    h = jnp.maximum(pre, 0.0)
    s = jnp.sum(h * sW2[...][None, :], axis=1)
    y = s.reshape(1, 1, TT) + g[...]
    col = t * TT + lax.broadcasted_iota(jnp.int32, (1, 1, TT), 2)
    out[...] = jnp.where((col == 0) | (col == T - 1), BIG, y)


def _scores(feat_seq, para, pW1a, pW1b, pb1, pW2, pb2, fWb, fb,
            fWa, emb_pad, Wc, A, P, sb1, sW2v, g3):
    return pl.pallas_call(
        _score_body,
        grid=(NT, B),
        in_specs=[
            pl.BlockSpec((1, TT, DIN), lambda t, b: (b, t, 0)),
            pl.BlockSpec((B, 2), lambda t, b: (0, 0)),
            pl.BlockSpec((2 * HID,), lambda t, b: (0,)),
            pl.BlockSpec((2 * HID,), lambda t, b: (0,)),
            pl.BlockSpec((2 * HID,), lambda t, b: (0,)),
            pl.BlockSpec((2 * HID, HID), lambda t, b: (0, 0)),
            pl.BlockSpec((HID,), lambda t, b: (0,)),
            pl.BlockSpec((HID, HID), lambda t, b: (0, 0)),
            pl.BlockSpec((HID,), lambda t, b: (0,)),
            pl.BlockSpec((DIN, HID), lambda t, b: (0, 0)),
            pl.BlockSpec((TT, HID), lambda t, b: (t, 0)),
            pl.BlockSpec((HID, HID), lambda t, b: (0, 0)),
            pl.BlockSpec((HID, HID), lambda t, b: (0, 0)),
            pl.BlockSpec((HID, HID), lambda t, b: (0, 0)),
            pl.BlockSpec((HID,), lambda t, b: (0,)),
            pl.BlockSpec((HID,), lambda t, b: (0,)),
            pl.BlockSpec((1, 1, TT), lambda t, b: (b * NT + t, 0, 0)),
        ],
        out_specs=pl.BlockSpec((1, 1, TT), lambda t, b: (b * NT + t, 0, 0)),
        out_shape=jax.ShapeDtypeStruct((B * NT, 1, TT), jnp.float32),
        scratch_shapes=[pltpu.VMEM((B, HID), jnp.float32),
                        pltpu.VMEM((TT, HID), jnp.float32),
                        pltpu.VMEM((DIN, 2 * HID), jnp.float32),
                        pltpu.VMEM((B, HID), jnp.float32)],
    )(feat_seq, para, pW1a, pW1b, pb1, pW2, pb2, fWb, fb,
      fWa, emb_pad, Wc, A, P, sb1, sW2v, g3)


def _topk_body(y_hbm, out_hbm, yv, ov):
    c = lax.axis_index("c")
    s = lax.axis_index("s")
    row = c * 16 + s

    @pl.when(row < B)
    def _():
        pltpu.sync_copy(y_hbm.at[row], yv)
        lanes = lax.broadcasted_iota(jnp.int32, (L,), 0)
        tk0 = jnp.full((L,), -BIG, jnp.float32)
        ti0 = jnp.zeros((L,), jnp.int32)

        def body(i, carry):
            tk, ti = carry
            v = yv[pl.ds(i * L, L)]
            vi = lanes + i * L
            vk, vix = plsc.sort_key_val(v, vi, descending=True)
            m = tk >= vk
            mk = jnp.where(m, tk, vk)
            mi = jnp.where(m, ti, vix)
            return tuple(plsc.sort_key_val(mk, mi))

        tk, ti = lax.fori_loop(0, T // L, body, (tk0, ti0))
        si, _ = plsc.sort_key_val(ti, ti)
        ov[...] = si
        pltpu.sync_copy(ov, out_hbm.at[row])


def _sc_topk(y):
    mesh = plsc.VectorSubcoreMesh(core_axis_name="c", subcore_axis_name="s")
    kern = pl.kernel(
        _topk_body,
        mesh=mesh,
        out_type=jax.ShapeDtypeStruct((B, K), jnp.int32),
        scratch_types=[pltpu.VMEM((T,), jnp.float32),
                       pltpu.VMEM((K,), jnp.int32)],
        compiler_params=pltpu.CompilerParams(needs_layout_passes=False),
    )
    return kern(y)


def kernel(feat_seq, para, pW1, pb1, pW2, pb2, fW, fb, emb_table, sW1, sb1, sW2, sb2):
    pW1a = pW1[0]
    pW1b = pW1[1]
    fWa = fW[:DIN]
    fWb = fW[DIN:]
    A = sW1[0:HID] + sW1[2 * HID:3 * HID]
    Wc = sW1[HID:2 * HID] - sW1[2 * HID:3 * HID]
    P = sW1[3 * HID:4 * HID] + sW1[4 * HID:4 * HID + 1]
    emb_pad = jnp.pad(emb_table, ((1, 1), (0, 0)))
    g = jax.random.gumbel(jax.random.key(42), (B, T - 2), jnp.float32)
    g3 = jnp.pad(g + sb2[0], ((0, 0), (1, 1))).reshape(B * NT, 1, TT)
    sW2v = sW2[:, 0]

    y3 = _scores(feat_seq, para, pW1a, pW1b, pb1, pW2, pb2, fWb, fb,
                 fWa, emb_pad, Wc, A, P, sb1, sW2v, g3)
    return _sc_topk(y3.reshape(B, T))


# speedup vs baseline: 1.2920x; 1.2920x over previous
import jax
import jax.numpy as jnp
from jax import lax
from jax.experimental import pallas as pl
from jax.experimental.pallas import tpu as pltpu
from jax.experimental.pallas import tpu_sc as plsc

B = 16
T = 2048
DIN = 256
HID = 256
K = 16
TT = 2048
NT = T // TT
BIG = 3.0e38
L = 16


def _split(w):
    hi = w.astype(jnp.bfloat16)
    lo = (w - hi.astype(jnp.float32)).astype(jnp.bfloat16)
    return hi, lo


def _dot3(ah, al, bh, bl):
    return (jnp.dot(ah, bh, preferred_element_type=jnp.float32)
            + jnp.dot(ah, bl, preferred_element_type=jnp.float32)
            + jnp.dot(al, bh, preferred_element_type=jnp.float32))


_HI = jax.lax.Precision.HIGHEST


def _score_body(feat, para, pW1a, pW1b, pb1, pW2, pb2, fWb, fb,
                fWa, emb, Wc, A, P, sb1, sW2, g, out, pconst, cblk,
                wch, wcl, Phs, Pls, pcA):
    t = pl.program_id(0)
    b = pl.program_id(1)

    @pl.when((t == 0) & (b == 0))
    def _():
        pv = para[...]
        h1 = (pv[:, 0:1] * pW1a[...][None, :]
              + pv[:, 1:2] * pW1b[...][None, :]
              + pb1[...][None, :])
        pe = (jnp.dot(jnp.maximum(h1, 0.0), pW2[...],
                      preferred_element_type=jnp.float32, precision=_HI)
              + pb2[...][None, :])
        pc = (jnp.dot(pe, fWb[...], preferred_element_type=jnp.float32,
                      precision=_HI)
              + fb[...][None, :])
        pconst[...] = pc
        fh, fl = _split(fWa[...])
        w2h, w2l = _split(jnp.dot(fWa[...], A[...],
                                  preferred_element_type=jnp.float32,
                                  precision=_HI))
        wch[:, :HID] = fh
        wcl[:, :HID] = fl
        wch[:, HID:] = w2h
        wcl[:, HID:] = w2l
        ph, pl_ = _split(P[...])
        Phs[...] = ph
        Pls[...] = pl_
        pcA[...] = jnp.dot(pc, A[...], preferred_element_type=jnp.float32,
                           precision=_HI)

    @pl.when(b == 0)
    def _():
        cblk[...] = (jnp.dot(emb[...], Wc[...],
                             preferred_element_type=jnp.float32, precision=_HI)
                     + sb1[...][None, :])

    x = feat[0]
    xh, xl = _split(x)
    r = _dot3(xh, xl, wch[...], wcl[...])
    fpm = r[:, :HID] + pconst[pl.ds(b, 1), :]
    prod = fpm * emb[...]
    qh, ql = _split(prod)
    pre = (_dot3(qh, ql, Phs[...], Pls[...])
           + r[:, HID:] + pcA[pl.ds(b, 1), :] + cblk[...])
    h = jnp.maximum(pre, 0.0)
    s = jnp.sum(h * sW2[...][None, :], axis=1)
    y = s.reshape(1, 1, TT) + g[...]
    col = t * TT + lax.broadcasted_iota(jnp.int32, (1, 1, TT), 2)
    out[...] = jnp.where((col == 0) | (col == T - 1), BIG, y)


def _scores(feat_seq, para, pW1a, pW1b, pb1, pW2, pb2, fWb, fb,
            fWa, emb_pad, Wc, A, P, sb1, sW2v, g3):
    return pl.pallas_call(
        _score_body,
        grid=(NT, B),
        in_specs=[
            pl.BlockSpec((1, TT, DIN), lambda t, b: (b, t, 0)),
            pl.BlockSpec((B, 2), lambda t, b: (0, 0)),
            pl.BlockSpec((2 * HID,), lambda t, b: (0,)),
            pl.BlockSpec((2 * HID,), lambda t, b: (0,)),
            pl.BlockSpec((2 * HID,), lambda t, b: (0,)),
            pl.BlockSpec((2 * HID, HID), lambda t, b: (0, 0)),
            pl.BlockSpec((HID,), lambda t, b: (0,)),
            pl.BlockSpec((HID, HID), lambda t, b: (0, 0)),
            pl.BlockSpec((HID,), lambda t, b: (0,)),
            pl.BlockSpec((DIN, HID), lambda t, b: (0, 0)),
            pl.BlockSpec((TT, HID), lambda t, b: (t, 0)),
            pl.BlockSpec((HID, HID), lambda t, b: (0, 0)),
            pl.BlockSpec((HID, HID), lambda t, b: (0, 0)),
            pl.BlockSpec((HID, HID), lambda t, b: (0, 0)),
            pl.BlockSpec((HID,), lambda t, b: (0,)),
            pl.BlockSpec((HID,), lambda t, b: (0,)),
            pl.BlockSpec((1, 1, TT), lambda t, b: (b * NT + t, 0, 0)),
        ],
        out_specs=pl.BlockSpec((1, 1, TT), lambda t, b: (b * NT + t, 0, 0)),
        out_shape=jax.ShapeDtypeStruct((B * NT, 1, TT), jnp.float32),
        scratch_shapes=[pltpu.VMEM((B, HID), jnp.float32),
                        pltpu.VMEM((TT, HID), jnp.float32),
                        pltpu.VMEM((DIN, 2 * HID), jnp.bfloat16),
                        pltpu.VMEM((DIN, 2 * HID), jnp.bfloat16),
                        pltpu.VMEM((HID, HID), jnp.bfloat16),
                        pltpu.VMEM((HID, HID), jnp.bfloat16),
                        pltpu.VMEM((B, HID), jnp.float32)],
    )(feat_seq, para, pW1a, pW1b, pb1, pW2, pb2, fWb, fb,
      fWa, emb_pad, Wc, A, P, sb1, sW2v, g3)


def _topk_body(y_hbm, out_hbm, yv, ov):
    c = lax.axis_index("c")
    s = lax.axis_index("s")
    row = c * 16 + s

    @pl.when(row < B)
    def _():
        pltpu.sync_copy(y_hbm.at[row], yv)
        lanes = lax.broadcasted_iota(jnp.int32, (L,), 0)
        tk0 = jnp.full((L,), -BIG, jnp.float32)
        ti0 = jnp.zeros((L,), jnp.int32)

        def body(i, carry):
            tk, ti = carry
            v = yv[pl.ds(i * L, L)]
            vi = lanes + i * L
            vk, vix = plsc.sort_key_val(v, vi, descending=True)
            m = tk >= vk
            mk = jnp.where(m, tk, vk)
            mi = jnp.where(m, ti, vix)
            return tuple(plsc.sort_key_val(mk, mi))

        tk, ti = lax.fori_loop(0, T // L, body, (tk0, ti0))
        si, _ = plsc.sort_key_val(ti, ti)
        ov[...] = si
        pltpu.sync_copy(ov, out_hbm.at[row])


def _sc_topk(y):
    mesh = plsc.VectorSubcoreMesh(core_axis_name="c", subcore_axis_name="s")
    kern = pl.kernel(
        _topk_body,
        mesh=mesh,
        out_type=jax.ShapeDtypeStruct((B, K), jnp.int32),
        scratch_types=[pltpu.VMEM((T,), jnp.float32),
                       pltpu.VMEM((K,), jnp.int32)],
        compiler_params=pltpu.CompilerParams(needs_layout_passes=False),
    )
    return kern(y)


def kernel(feat_seq, para, pW1, pb1, pW2, pb2, fW, fb, emb_table, sW1, sb1, sW2, sb2):
    pW1a = pW1[0]
    pW1b = pW1[1]
    fWa = fW[:DIN]
    fWb = fW[DIN:]
    A = sW1[0:HID] + sW1[2 * HID:3 * HID]
    Wc = sW1[HID:2 * HID] - sW1[2 * HID:3 * HID]
    P = sW1[3 * HID:4 * HID] + sW1[4 * HID:4 * HID + 1]
    emb_pad = jnp.pad(emb_table, ((1, 1), (0, 0)))
    g = jax.random.gumbel(jax.random.key(42), (B, T - 2), jnp.float32)
    g3 = jnp.pad(g + sb2[0], ((0, 0), (1, 1))).reshape(B * NT, 1, TT)
    sW2v = sW2[:, 0]

    y3 = _scores(feat_seq, para, pW1a, pW1b, pb1, pW2, pb2, fWb, fb,
                 fWa, emb_pad, Wc, A, P, sb1, sW2v, g3)
    return _sc_topk(y3.reshape(B, T))


# speedup vs baseline: 2.4521x; 1.8980x over previous
import jax
import jax.numpy as jnp
from jax import lax
from jax.experimental import pallas as pl
from jax.experimental.pallas import tpu as pltpu
from jax.experimental.pallas import tpu_sc as plsc

B = 16
T = 2048
DIN = 256
HID = 256
K = 16
TT = 2048
NT = T // TT
BIG = 3.0e38
L = 16


def _score_body(feat, para, pW1a, pW1b, pb1, pW2, pb2, fWb, fb,
                fWa, emb, Wc, A, P, sb1, sW2, g, out, pconst, cblk,
                wcat, pcA):
    t = pl.program_id(0)
    b = pl.program_id(1)

    @pl.when((t == 0) & (b == 0))
    def _():
        pv = para[...]
        h1 = (pv[:, 0:1] * pW1a[...][None, :]
              + pv[:, 1:2] * pW1b[...][None, :]
              + pb1[...][None, :])
        pe = (jnp.dot(jnp.maximum(h1, 0.0), pW2[...],
                      preferred_element_type=jnp.float32)
              + pb2[...][None, :])
        pc = (jnp.dot(pe, fWb[...], preferred_element_type=jnp.float32)
              + fb[...][None, :])
        pconst[...] = pc
        wcat[:, :HID] = fWa[...]
        wcat[:, HID:] = jnp.dot(fWa[...], A[...],
                                preferred_element_type=jnp.float32)
        pcA[...] = jnp.dot(pc, A[...], preferred_element_type=jnp.float32)

    @pl.when(b == 0)
    def _():
        cblk[...] = (jnp.dot(emb[...], Wc[...],
                             preferred_element_type=jnp.float32)
                     + sb1[...][None, :])

    pc = pconst[pl.ds(b, 1), :]
    pca = pcA[pl.ds(b, 1), :]
    w = wcat[...]
    Pm = P[...]
    sw = sW2[...]
    NCH = 4
    CH = TT // NCH
    for c in range(NCH):
        x = feat[0, pl.ds(c * CH, CH), :]
        ec = emb[pl.ds(c * CH, CH), :]
        r = jnp.dot(x, w, preferred_element_type=jnp.float32)
        fpm = r[:, :HID] + pc
        prod = fpm * ec
        pre = (jnp.dot(prod, Pm, preferred_element_type=jnp.float32)
               + r[:, HID:] + pca + cblk[pl.ds(c * CH, CH), :])
        h = jnp.maximum(pre, 0.0)
        s = jnp.sum(h * sw[None, :], axis=1)
        y = s.reshape(1, 1, CH) + g[0, 0, pl.ds(c * CH, CH)].reshape(1, 1, CH)
        col = (t * TT + c * CH
               + lax.broadcasted_iota(jnp.int32, (1, 1, CH), 2))
        out[0, 0, pl.ds(c * CH, CH)] = jnp.where(
            (col == 0) | (col == T - 1), BIG, y)[0, 0]


def _scores(feat_seq, para, pW1a, pW1b, pb1, pW2, pb2, fWb, fb,
            fWa, emb_pad, Wc, A, P, sb1, sW2v, g3):
    return pl.pallas_call(
        _score_body,
        grid=(NT, B),
        in_specs=[
            pl.BlockSpec((1, TT, DIN), lambda t, b: (b, t, 0)),
            pl.BlockSpec((B, 2), lambda t, b: (0, 0)),
            pl.BlockSpec((2 * HID,), lambda t, b: (0,)),
            pl.BlockSpec((2 * HID,), lambda t, b: (0,)),
            pl.BlockSpec((2 * HID,), lambda t, b: (0,)),
            pl.BlockSpec((2 * HID, HID), lambda t, b: (0, 0)),
            pl.BlockSpec((HID,), lambda t, b: (0,)),
            pl.BlockSpec((HID, HID), lambda t, b: (0, 0)),
            pl.BlockSpec((HID,), lambda t, b: (0,)),
            pl.BlockSpec((DIN, HID), lambda t, b: (0, 0)),
            pl.BlockSpec((TT, HID), lambda t, b: (t, 0)),
            pl.BlockSpec((HID, HID), lambda t, b: (0, 0)),
            pl.BlockSpec((HID, HID), lambda t, b: (0, 0)),
            pl.BlockSpec((HID, HID), lambda t, b: (0, 0)),
            pl.BlockSpec((HID,), lambda t, b: (0,)),
            pl.BlockSpec((HID,), lambda t, b: (0,)),
            pl.BlockSpec((1, 1, TT), lambda t, b: (b * NT + t, 0, 0)),
        ],
        out_specs=pl.BlockSpec((1, 1, TT), lambda t, b: (b * NT + t, 0, 0)),
        out_shape=jax.ShapeDtypeStruct((B * NT, 1, TT), jnp.float32),
        scratch_shapes=[pltpu.VMEM((B, HID), jnp.float32),
                        pltpu.VMEM((TT, HID), jnp.float32),
                        pltpu.VMEM((DIN, 2 * HID), jnp.float32),
                        pltpu.VMEM((B, HID), jnp.float32)],
    )(feat_seq, para, pW1a, pW1b, pb1, pW2, pb2, fWb, fb,
      fWa, emb_pad, Wc, A, P, sb1, sW2v, g3)


def _topk_body(y_hbm, out_hbm, yv, ov):
    c = lax.axis_index("c")
    s = lax.axis_index("s")
    row = c * 16 + s

    @pl.when(row < B)
    def _():
        pltpu.sync_copy(y_hbm.at[row], yv)
        lanes = lax.broadcasted_iota(jnp.int32, (L,), 0)
        tk0 = jnp.full((L,), -BIG, jnp.float32)
        ti0 = jnp.zeros((L,), jnp.int32)

        def body(i, carry):
            tk, ti = carry
            v = yv[pl.ds(i * L, L)]
            vi = lanes + i * L
            vk, vix = plsc.sort_key_val(v, vi, descending=True)
            m = tk >= vk
            mk = jnp.where(m, tk, vk)
            mi = jnp.where(m, ti, vix)
            return tuple(plsc.sort_key_val(mk, mi))

        tk, ti = lax.fori_loop(0, T // L, body, (tk0, ti0))
        si, _ = plsc.sort_key_val(ti, ti)
        ov[...] = si
        pltpu.sync_copy(ov, out_hbm.at[row])


def _sc_topk(y):
    mesh = plsc.VectorSubcoreMesh(core_axis_name="c", subcore_axis_name="s")
    kern = pl.kernel(
        _topk_body,
        mesh=mesh,
        out_type=jax.ShapeDtypeStruct((B, K), jnp.int32),
        scratch_types=[pltpu.VMEM((T,), jnp.float32),
                       pltpu.VMEM((K,), jnp.int32)],
        compiler_params=pltpu.CompilerParams(needs_layout_passes=False),
    )
    return kern(y)


def kernel(feat_seq, para, pW1, pb1, pW2, pb2, fW, fb, emb_table, sW1, sb1, sW2, sb2):
    pW1a = pW1[0]
    pW1b = pW1[1]
    fWa = fW[:DIN]
    fWb = fW[DIN:]
    A = sW1[0:HID] + sW1[2 * HID:3 * HID]
    Wc = sW1[HID:2 * HID] - sW1[2 * HID:3 * HID]
    P = sW1[3 * HID:4 * HID] + sW1[4 * HID:4 * HID + 1]
    emb_pad = jnp.pad(emb_table, ((1, 1), (0, 0)))
    g = jax.random.gumbel(jax.random.key(42), (B, T - 2), jnp.float32)
    g3 = jnp.pad(g + sb2[0], ((0, 0), (1, 1))).reshape(B * NT, 1, TT)
    sW2v = sW2[:, 0]

    y3 = _scores(feat_seq, para, pW1a, pW1b, pb1, pW2, pb2, fWb, fb,
                 fWa, emb_pad, Wc, A, P, sb1, sW2v, g3)
    return _sc_topk(y3.reshape(B, T))


# speedup vs baseline: 2.7063x; 1.1037x over previous
import jax
import jax.numpy as jnp
from jax import lax
from jax.experimental import pallas as pl
from jax.experimental.pallas import tpu as pltpu
from jax.experimental.pallas import tpu_sc as plsc

B = 16
T = 2048
DIN = 256
HID = 256
K = 16
TT = 2048
NT = T // TT
BIG = 3.0e38
L = 16


def _score_body(feat, para, pW1a, pW1b, pb1, pW2, pb2, fWb, fb,
                fWa, emb, Wc, A, P, sb1, sW2, g, out, pconst, cblk,
                wcat, pcA):
    t = pl.program_id(0)
    b = pl.program_id(1)

    @pl.when((t == 0) & (b == 0))
    def _():
        pv = para[...]
        h1 = (pv[:, 0:1] * pW1a[...][None, :]
              + pv[:, 1:2] * pW1b[...][None, :]
              + pb1[...][None, :])
        pe = (jnp.dot(jnp.maximum(h1, 0.0), pW2[...],
                      preferred_element_type=jnp.float32)
              + pb2[...][None, :])
        pc = (jnp.dot(pe, fWb[...], preferred_element_type=jnp.float32)
              + fb[...][None, :])
        pconst[...] = pc
        wcat[:, :HID] = fWa[...]
        wcat[:, HID:] = jnp.dot(fWa[...], A[...],
                                preferred_element_type=jnp.float32)
        pcA[...] = jnp.dot(pc, A[...], preferred_element_type=jnp.float32)

    @pl.when(b == 0)
    def _():
        cblk[...] = (jnp.dot(emb[...], Wc[...],
                             preferred_element_type=jnp.float32)
                     + sb1[...][None, :])

    pc = pconst[pl.ds(b, 1), :]
    pca = pcA[pl.ds(b, 1), :]
    w = wcat[...]
    Pm = P[...]
    sw = sW2[...]
    NCH = 4
    CH = TT // NCH
    for c in range(NCH):
        x = feat[0, pl.ds(c * CH, CH), :]
        ec = emb[pl.ds(c * CH, CH), :]
        r = jnp.dot(x, w, preferred_element_type=jnp.float32)
        fpm = r[:, :HID] + pc
        prod = fpm * ec
        pre = (jnp.dot(prod, Pm, preferred_element_type=jnp.float32)
               + r[:, HID:] + pca + cblk[pl.ds(c * CH, CH), :])
        h = jnp.maximum(pre, 0.0)
        s = jnp.sum(h * sw[None, :], axis=1)
        out[0, 0, pl.ds(c * CH, CH)] = s + g[0, 0, pl.ds(c * CH, CH)]


def _scores(feat_seq, para, pW1a, pW1b, pb1, pW2, pb2, fWb, fb,
            fWa, emb_pad, Wc, A, P, sb1, sW2v, g3):
    return pl.pallas_call(
        _score_body,
        grid=(NT, B),
        in_specs=[
            pl.BlockSpec((1, TT, DIN), lambda t, b: (b, t, 0)),
            pl.BlockSpec((B, 2), lambda t, b: (0, 0)),
            pl.BlockSpec((2 * HID,), lambda t, b: (0,)),
            pl.BlockSpec((2 * HID,), lambda t, b: (0,)),
            pl.BlockSpec((2 * HID,), lambda t, b: (0,)),
            pl.BlockSpec((2 * HID, HID), lambda t, b: (0, 0)),
            pl.BlockSpec((HID,), lambda t, b: (0,)),
            pl.BlockSpec((HID, HID), lambda t, b: (0, 0)),
            pl.BlockSpec((HID,), lambda t, b: (0,)),
            pl.BlockSpec((DIN, HID), lambda t, b: (0, 0)),
            pl.BlockSpec((TT, HID), lambda t, b: (t, 0)),
            pl.BlockSpec((HID, HID), lambda t, b: (0, 0)),
            pl.BlockSpec((HID, HID), lambda t, b: (0, 0)),
            pl.BlockSpec((HID, HID), lambda t, b: (0, 0)),
            pl.BlockSpec((HID,), lambda t, b: (0,)),
            pl.BlockSpec((HID,), lambda t, b: (0,)),
            pl.BlockSpec((1, 1, TT), lambda t, b: (b * NT + t, 0, 0)),
        ],
        out_specs=pl.BlockSpec((1, 1, TT), lambda t, b: (b * NT + t, 0, 0)),
        out_shape=jax.ShapeDtypeStruct((B * NT, 1, TT), jnp.float32),
        scratch_shapes=[pltpu.VMEM((B, HID), jnp.float32),
                        pltpu.VMEM((TT, HID), jnp.float32),
                        pltpu.VMEM((DIN, 2 * HID), jnp.float32),
                        pltpu.VMEM((B, HID), jnp.float32)],
    )(feat_seq, para, pW1a, pW1b, pb1, pW2, pb2, fWb, fb,
      fWa, emb_pad, Wc, A, P, sb1, sW2v, g3)


def _topk_body(y_hbm, out_hbm, yv, ov):
    c = lax.axis_index("c")
    s = lax.axis_index("s")
    row = c * 16 + s

    @pl.when(row < B)
    def _():
        pltpu.sync_copy(y_hbm.at[row], yv)
        lanes = lax.broadcasted_iota(jnp.int32, (L,), 0)
        tk0 = jnp.full((L,), -BIG, jnp.float32)
        ti0 = jnp.zeros((L,), jnp.int32)

        def body(i, carry):
            tk, ti = carry
            v = yv[pl.ds(i * L, L)]
            vi = lanes + i * L
            vk, vix = plsc.sort_key_val(v, vi, descending=True)
            m = tk >= vk
            mk = jnp.where(m, tk, vk)
            mi = jnp.where(m, ti, vix)
            return tuple(plsc.sort_key_val(mk, mi))

        tk, ti = lax.fori_loop(0, T // L, body, (tk0, ti0))
        si, _ = plsc.sort_key_val(ti, ti)
        ov[...] = si
        pltpu.sync_copy(ov, out_hbm.at[row])


def _sc_topk(y):
    mesh = plsc.VectorSubcoreMesh(core_axis_name="c", subcore_axis_name="s")
    kern = pl.kernel(
        _topk_body,
        mesh=mesh,
        out_type=jax.ShapeDtypeStruct((B, K), jnp.int32),
        scratch_types=[pltpu.VMEM((T,), jnp.float32),
                       pltpu.VMEM((K,), jnp.int32)],
        compiler_params=pltpu.CompilerParams(needs_layout_passes=False),
    )
    return kern(y)


def kernel(feat_seq, para, pW1, pb1, pW2, pb2, fW, fb, emb_table, sW1, sb1, sW2, sb2):
    pW1a = pW1[0]
    pW1b = pW1[1]
    fWa = fW[:DIN]
    fWb = fW[DIN:]
    A = sW1[0:HID] + sW1[2 * HID:3 * HID]
    Wc = sW1[HID:2 * HID] - sW1[2 * HID:3 * HID]
    P = sW1[3 * HID:4 * HID] + sW1[4 * HID:4 * HID + 1]
    emb_pad = jnp.pad(emb_table, ((1, 1), (0, 0)))
    g = jax.random.gumbel(jax.random.key(42), (B, T - 2), jnp.float32)
    g3 = jnp.pad(g + sb2[0], ((0, 0), (1, 1)),
                 constant_values=BIG).reshape(B * NT, 1, TT)
    sW2v = sW2[:, 0]

    y3 = _scores(feat_seq, para, pW1a, pW1b, pb1, pW2, pb2, fWb, fb,
                 fWa, emb_pad, Wc, A, P, sb1, sW2v, g3)
    return _sc_topk(y3.reshape(B, T))


# speedup vs baseline: 2.8264x; 1.0444x over previous
import jax
import jax.numpy as jnp
from jax import lax
from jax.experimental import pallas as pl
from jax.experimental.pallas import tpu as pltpu
from jax.experimental.pallas import tpu_sc as plsc

B = 16
T = 2048
DIN = 256
HID = 256
K = 16
TT = 2048
NT = T // TT
BIG = 3.0e38
L = 16


def _score_body(feat, para, pW1a, pW1b, pb1, pW2, pb2, fWb, fb,
                fWa, emb, Wc, A, P, sb1, sW2, g, out, pconst, cblk,
                wcat, pcA):
    t = pl.program_id(0)
    b = pl.program_id(1)

    @pl.when((t == 0) & (b == 0))
    def _():
        pv = para[...]
        h1 = (pv[:, 0:1] * pW1a[...][None, :]
              + pv[:, 1:2] * pW1b[...][None, :]
              + pb1[...][None, :])
        pe = (jnp.dot(jnp.maximum(h1, 0.0), pW2[...],
                      preferred_element_type=jnp.float32)
              + pb2[...][None, :])
        pc = (jnp.dot(pe, fWb[...], preferred_element_type=jnp.float32)
              + fb[...][None, :])
        pconst[...] = pc
        wcat[:, :HID] = fWa[...]
        wcat[:, HID:] = jnp.dot(fWa[...], A[...],
                                preferred_element_type=jnp.float32)
        pcA[...] = jnp.dot(pc, A[...], preferred_element_type=jnp.float32)

    @pl.when(b == 0)
    def _():
        cblk[...] = (jnp.dot(emb[...], Wc[...],
                             preferred_element_type=jnp.float32)
                     + sb1[...][None, :])

    pc = pconst[pl.ds(b, 1), :]
    pca = pcA[pl.ds(b, 1), :]
    w = wcat[...]
    Pm = P[...]
    sw = sW2[...]
    NCH = 8
    CH = TT // NCH
    for c in range(NCH):
        x = feat[0, pl.ds(c * CH, CH), :]
        ec = emb[pl.ds(c * CH, CH), :]
        r = jnp.dot(x, w, preferred_element_type=jnp.float32)
        fpm = r[:, :HID] + pc
        prod = fpm * ec
        pre = (jnp.dot(prod, Pm, preferred_element_type=jnp.float32)
               + r[:, HID:] + pca + cblk[pl.ds(c * CH, CH), :])
        h = jnp.maximum(pre, 0.0)
        s = jnp.sum(h * sw[None, :], axis=1)
        out[0, 0, pl.ds(c * CH, CH)] = s + g[0, 0, pl.ds(c * CH, CH)]


def _scores(feat_seq, para, pW1a, pW1b, pb1, pW2, pb2, fWb, fb,
            fWa, emb_pad, Wc, A, P, sb1, sW2v, g3):
    return pl.pallas_call(
        _score_body,
        grid=(NT, B),
        in_specs=[
            pl.BlockSpec((1, TT, DIN), lambda t, b: (b, t, 0)),
            pl.BlockSpec((B, 2), lambda t, b: (0, 0)),
            pl.BlockSpec((2 * HID,), lambda t, b: (0,)),
            pl.BlockSpec((2 * HID,), lambda t, b: (0,)),
            pl.BlockSpec((2 * HID,), lambda t, b: (0,)),
            pl.BlockSpec((2 * HID, HID), lambda t, b: (0, 0)),
            pl.BlockSpec((HID,), lambda t, b: (0,)),
            pl.BlockSpec((HID, HID), lambda t, b: (0, 0)),
            pl.BlockSpec((HID,), lambda t, b: (0,)),
            pl.BlockSpec((DIN, HID), lambda t, b: (0, 0)),
            pl.BlockSpec((TT, HID), lambda t, b: (t, 0)),
            pl.BlockSpec((HID, HID), lambda t, b: (0, 0)),
            pl.BlockSpec((HID, HID), lambda t, b: (0, 0)),
            pl.BlockSpec((HID, HID), lambda t, b: (0, 0)),
            pl.BlockSpec((HID,), lambda t, b: (0,)),
            pl.BlockSpec((HID,), lambda t, b: (0,)),
            pl.BlockSpec((1, 1, TT), lambda t, b: (b * NT + t, 0, 0)),
        ],
        out_specs=pl.BlockSpec((1, 1, TT), lambda t, b: (b * NT + t, 0, 0)),
        out_shape=jax.ShapeDtypeStruct((B * NT, 1, TT), jnp.float32),
        scratch_shapes=[pltpu.VMEM((B, HID), jnp.float32),
                        pltpu.VMEM((TT, HID), jnp.float32),
                        pltpu.VMEM((DIN, 2 * HID), jnp.float32),
                        pltpu.VMEM((B, HID), jnp.float32)],
    )(feat_seq, para, pW1a, pW1b, pb1, pW2, pb2, fWb, fb,
      fWa, emb_pad, Wc, A, P, sb1, sW2v, g3)


def _topk_body(y_hbm, out_hbm, yv, ov):
    c = lax.axis_index("c")
    s = lax.axis_index("s")
    row = c * 16 + s

    @pl.when(row < B)
    def _():
        pltpu.sync_copy(y_hbm.at[row], yv)
        lanes = lax.broadcasted_iota(jnp.int32, (L,), 0)
        tk0 = jnp.full((L,), -BIG, jnp.float32)
        ti0 = jnp.zeros((L,), jnp.int32)

        def body(i, carry):
            tk, ti = carry
            v = yv[pl.ds(i * L, L)]
            vi = lanes + i * L
            vk, vix = plsc.sort_key_val(v, vi, descending=True)
            m = tk >= vk
            mk = jnp.where(m, tk, vk)
            mi = jnp.where(m, ti, vix)
            return tuple(plsc.sort_key_val(mk, mi))

        tk, ti = lax.fori_loop(0, T // L, body, (tk0, ti0))
        si, _ = plsc.sort_key_val(ti, ti)
        ov[...] = si
        pltpu.sync_copy(ov, out_hbm.at[row])


def _sc_topk(y):
    mesh = plsc.VectorSubcoreMesh(core_axis_name="c", subcore_axis_name="s")
    kern = pl.kernel(
        _topk_body,
        mesh=mesh,
        out_type=jax.ShapeDtypeStruct((B, K), jnp.int32),
        scratch_types=[pltpu.VMEM((T,), jnp.float32),
                       pltpu.VMEM((K,), jnp.int32)],
        compiler_params=pltpu.CompilerParams(needs_layout_passes=False),
    )
    return kern(y)


def kernel(feat_seq, para, pW1, pb1, pW2, pb2, fW, fb, emb_table, sW1, sb1, sW2, sb2):
    pW1a = pW1[0]
    pW1b = pW1[1]
    fWa = fW[:DIN]
    fWb = fW[DIN:]
    A = sW1[0:HID] + sW1[2 * HID:3 * HID]
    Wc = sW1[HID:2 * HID] - sW1[2 * HID:3 * HID]
    P = sW1[3 * HID:4 * HID] + sW1[4 * HID:4 * HID + 1]
    emb_pad = jnp.pad(emb_table, ((1, 1), (0, 0)))
    g = jax.random.gumbel(jax.random.key(42), (B, T - 2), jnp.float32)
    g3 = jnp.pad(g + sb2[0], ((0, 0), (1, 1)),
                 constant_values=BIG).reshape(B * NT, 1, TT)
    sW2v = sW2[:, 0]

    y3 = _scores(feat_seq, para, pW1a, pW1b, pb1, pW2, pb2, fWb, fb,
                 fWa, emb_pad, Wc, A, P, sb1, sW2v, g3)
    return _sc_topk(y3.reshape(B, T))
